# baseline (speedup 1.0000x reference)
import functools
import math

import jax
import jax.numpy as jnp
from jax import lax
from jax.experimental import pallas as pl
from jax.experimental.pallas import tpu as pltpu
from jax.experimental.pallas import tpu_sc as plsc

D_MODEL = 64
SCALE = math.sqrt(D_MODEL)

_NC = 2
_NS = 16
_NW = _NC * _NS
CHUNK = 128


@functools.lru_cache(maxsize=None)
def _make_kernel(B: int):
    n_chunks = B // (_NW * CHUNK)
    mesh = plsc.VectorSubcoreMesh(core_axis_name="c", subcore_axis_name="s")

    @functools.partial(
        pl.kernel,
        mesh=mesh,
        out_type=jax.ShapeDtypeStruct((B, D_MODEL), jnp.float32),
        scratch_types=[
            pltpu.VMEM((n_chunks, CHUNK), jnp.int32),
            pltpu.VMEM((CHUNK, D_MODEL), jnp.float32),
            pltpu.SemaphoreType.DMA,
        ],
        compiler_params=pltpu.CompilerParams(use_tc_tiling_on_sc=False),
    )
    def k(x_hbm, lut_hbm, out_hbm, idx_v, rows_v, sem):
        wid = lax.axis_index("s") * _NC + lax.axis_index("c")
        pltpu.sync_copy(x_hbm.at[wid], idx_v)

        def chunk_body(j, carry):
            pltpu.async_copy(lut_hbm.at[idx_v.at[j]], rows_v, sem).wait()

            def scale_body(r, carry2):
                for t in range(D_MODEL // 16):
                    sl = pl.ds(t * 16, 16)
                    rows_v[r, sl] = rows_v[r, sl] * SCALE
                return carry2

            lax.fori_loop(0, CHUNK, scale_body, 0)
            base = (wid * n_chunks + j) * CHUNK
            pltpu.sync_copy(rows_v, out_hbm.at[pl.ds(base, CHUNK)])
            return carry

        lax.fori_loop(0, n_chunks, chunk_body, 0)

    return k


def kernel(x, lut):
    b0, b1 = x.shape
    B = b0 * b1
    xr = x.astype(jnp.int32).reshape(_NW, B // (_NW * CHUNK), CHUNK)
    out = _make_kernel(B)(xr, lut)
    return out.reshape(b0, b1, D_MODEL)


# speedup vs baseline: 1.0943x; 1.0943x over previous
import functools
import math

import jax
import jax.numpy as jnp
from jax import lax
from jax.experimental import pallas as pl
from jax.experimental.pallas import tpu as pltpu
from jax.experimental.pallas import tpu_sc as plsc

D_MODEL = 64
SCALE = math.sqrt(D_MODEL)

_NC = 2
_NS = 16
_NW = _NC * _NS
CHUNK = 128
NBUF = 4


@functools.lru_cache(maxsize=None)
def _make_kernel(B: int):
    n_chunks = B // (_NW * CHUNK)
    n_groups = n_chunks // NBUF
    mesh = plsc.VectorSubcoreMesh(core_axis_name="c", subcore_axis_name="s")

    @functools.partial(
        pl.kernel,
        mesh=mesh,
        out_type=jax.ShapeDtypeStruct((B, D_MODEL), jnp.float32),
        scratch_types=(
            [pltpu.VMEM((n_chunks, CHUNK), jnp.int32)]
            + [pltpu.VMEM((NBUF, CHUNK, D_MODEL), jnp.float32)] * 2
            + [pltpu.SemaphoreType.DMA] * (2 * NBUF)
        ),
        compiler_params=pltpu.CompilerParams(use_tc_tiling_on_sc=False),
    )
    def k(x_hbm, lut_hbm, out_hbm, idx_v, gbuf, sbuf, *sems):
        gsem = sems[:NBUF]
        ssem = sems[NBUF:]
        wid = lax.axis_index("s") * _NC + lax.axis_index("c")
        pltpu.sync_copy(x_hbm.at[wid], idx_v)
        row0 = wid * n_chunks * CHUNK

        def start_gather(j, b):
            pltpu.async_copy(lut_hbm.at[idx_v.at[j]], gbuf.at[b], gsem[b])

        def wait_gather(b):
            pltpu.make_async_copy(lut_hbm.at[idx_v.at[0]], gbuf.at[b],
                                  gsem[b]).wait()

        for b in range(NBUF):
            start_gather(b, b)

        def group_body(g, carry):
            for b in range(NBUF):
                j = g * NBUF + b
                wait_gather(b)

                @pl.when(g > 0)
                def _():
                    pltpu.make_async_copy(
                        sbuf.at[b], out_hbm.at[pl.ds(row0, CHUNK)],
                        ssem[b]).wait()

                def scale_body(r, carry2):
                    for t in range(D_MODEL // 16):
                        sl = pl.ds(t * 16, 16)
                        sbuf[b, r, sl] = gbuf[b, r, sl] * SCALE
                    return carry2

                lax.fori_loop(0, CHUNK, scale_body, 0, unroll=2)

                @pl.when(j + NBUF < n_chunks)
                def _():
                    start_gather(j + NBUF, b)

                pltpu.async_copy(sbuf.at[b],
                                 out_hbm.at[pl.ds(row0 + j * CHUNK, CHUNK)],
                                 ssem[b])
            return carry

        lax.fori_loop(0, n_groups, group_body, 0)

        for b in range(NBUF):
            pltpu.make_async_copy(sbuf.at[b], out_hbm.at[pl.ds(row0, CHUNK)],
                                  ssem[b]).wait()

    return k


def kernel(x, lut):
    b0, b1 = x.shape
    B = b0 * b1
    xr = x.astype(jnp.int32).reshape(_NW, B // (_NW * CHUNK), CHUNK)
    out = _make_kernel(B)(xr, lut)
    return out.reshape(b0, b1, D_MODEL)
